# Initial kernel scaffold; baseline (speedup 1.0000x reference)
import jax
import jax.numpy as jnp
from jax.experimental import pallas as pl


def kernel(x, edge_index, W1l, W1r, b1, W2l, W2r, b2, W3l, W3r, b3, Wlin1, blin1, Wlin2, blin2):
    raise NotImplementedError("write your pallas kernel here")


# speedup vs baseline: 1.2610x; 1.2610x over previous
import functools

import jax
import jax.numpy as jnp
from jax import lax
from jax.experimental import pallas as pl
from jax.experimental.pallas import tpu as pltpu
from jax.experimental.pallas import tpu_sc as plsc

N = 10000
E = 320000
D = 128
H = 128
C = 64

NC = 2
NS = 16
NW = NC * NS
RNG = 320
NP = NW * RNG
TRASH = RNG
BLKE = 512
CAP = E // BLKE + 3
EROWS = E // 128
SCCH = 4
NCHUNK = EROWS // SCCH
PEND = 1536

_MESH = plsc.VectorSubcoreMesh(
    core_axis_name="c", subcore_axis_name="s", num_cores=NC, num_subcores=NS
)


def _tile_id():
    return lax.axis_index("c") * NS + lax.axis_index("s")


def _compact_body(src_hbm, dst_hbm, list_hbm, nblk_hbm,
                  s0, d0, s1, d1, psrc, pdl, bounce, nv, semi):
    t = _tile_id()
    lo = t * RNG

    def maybe_flush(pos, nblk):
        do = pos >= 512

        @pl.when(do)
        def _():
            flush_io(nblk)

        return (jnp.where(do, pos - 512, pos),
                jnp.where(do, nblk + 1, nblk))

    def flush(pos, nblk):
        flush_io(nblk)
        return pos - 512, nblk + 1

    def flush_io(nblk):
        for i in range(4):
            pltpu.sync_copy(psrc.at[pl.ds(i * 128, 128)],
                            list_hbm.at[t, nblk, 0, i])
            pltpu.sync_copy(pdl.at[pl.ds(i * 128, 128)],
                            list_hbm.at[t, nblk, 1, i])
        for p in (psrc, pdl):
            for i in range(64):
                p[pl.ds(i * 16, 16)] = p[pl.ds(512 + i * 16, 16)]

    lanes = lax.iota(jnp.int32, 16)

    def scan_chunk(sbuf, dbuf, pos):
        for r in range(SCCH):
            for g in range(8):
                src16 = sbuf[r, g * 16:(g + 1) * 16]
                dst16 = dbuf[r, g * 16:(g + 1) * 16]
                u = dst16 - lo
                m32 = ((u | (RNG - 1 - u)) >> 31) + 1
                cs = plsc.cumsum(m32)
                good = pos + cs - m32
                positions = good * m32 + (PEND - 16 + lanes) * (1 - m32)
                plsc.store_scatter(psrc, [positions], src16)
                plsc.store_scatter(pdl, [positions], u)
                pos = pos + cs[15]
        return pos

    def stage(ch, sbuf, dbuf):
        pltpu.async_copy(src_hbm.at[pl.ds(ch * SCCH, SCCH)], sbuf, semi)
        pltpu.async_copy(dst_hbm.at[pl.ds(ch * SCCH, SCCH)], dbuf, semi)

    def stage_wait(ch, sbuf, dbuf):
        pltpu.make_async_copy(src_hbm.at[pl.ds(ch * SCCH, SCCH)], sbuf,
                              semi).wait()
        pltpu.make_async_copy(dst_hbm.at[pl.ds(ch * SCCH, SCCH)], dbuf,
                              semi).wait()

    stage(0, s0, d0)

    def body(i, carry):
        pos, nblk = carry
        ch0 = i * 2
        ch1 = ch0 + 1
        stage_wait(ch0, s0, d0)

        @pl.when(ch1 < NCHUNK)
        def _():
            stage(ch1, s1, d1)

        pos = scan_chunk(s0, d0, pos)
        pos, nblk = maybe_flush(pos, nblk)

        @pl.when(ch1 < NCHUNK)
        def _():
            stage_wait(ch1, s1, d1)

        @pl.when(ch0 + 2 < NCHUNK)
        def _():
            stage(ch0 + 2, s0, d0)

        pos1 = scan_chunk(s1, d1, pos)
        pos = jnp.where(ch1 < NCHUNK, pos1, pos)
        pos, nblk = maybe_flush(pos, nblk)
        return pos, nblk

    npairs = (NCHUNK + 1) // 2
    pos, nblk = lax.fori_loop(0, npairs, body, (jnp.int32(0), jnp.int32(0)))

    trash_src = jnp.zeros((16,), jnp.int32)
    trash_dl = jnp.full((16,), TRASH, jnp.int32)
    for i in range(32):
        psrc[pl.ds(pos + i * 16, 16)] = trash_src
        pdl[pl.ds(pos + i * 16, 16)] = trash_dl
    pos, nblk = flush(pos + 512, nblk)

    for i in range(32):
        psrc[pl.ds(i * 16, 16)] = trash_src
        pdl[pl.ds(i * 16, 16)] = trash_dl

    odd = nblk % 2 == 1

    @pl.when(odd)
    def _():
        flush_io(nblk)

    nblk = jnp.where(odd, nblk + 1, nblk)

    nv[...] = jnp.full((16,), 0, jnp.int32) + nblk
    pltpu.sync_copy(nv, nblk_hbm.at[t])


_compact = pl.kernel(
    _compact_body,
    out_type=(
        jax.ShapeDtypeStruct((NW, CAP, 2, 4, 128), jnp.int32),
        jax.ShapeDtypeStruct((NW, 16), jnp.int32),
    ),
    mesh=_MESH,
    compiler_params=pltpu.CompilerParams(needs_layout_passes=False),
    scratch_types=[
        pltpu.VMEM((SCCH, 128), jnp.int32),
        pltpu.VMEM((SCCH, 128), jnp.int32),
        pltpu.VMEM((SCCH, 128), jnp.int32),
        pltpu.VMEM((SCCH, 128), jnp.int32),
        pltpu.VMEM((PEND,), jnp.int32),
        pltpu.VMEM((PEND,), jnp.int32),
        pltpu.VMEM((1024,), jnp.int32),
        pltpu.VMEM((16,), jnp.int32),
        pltpu.SemaphoreType.DMA,
    ],
)


def _agg_body(h_hbm, list_hbm, nblk_hbm, out_hbm,
              ca, cb, rows0, rows1, nv, acc, semi, semg0, semg1):
    t = _tile_id()
    lo = pl.multiple_of(t * RNG, 8)

    pltpu.sync_copy(nblk_hbm.at[t], nv)
    nblk = nv[...][0]

    zero = jnp.zeros((16,), jnp.float32)

    def zb(i, _):
        for g in range(8):
            acc[i, g * 16:(g + 1) * 16] = zero
        return 0
    lax.fori_loop(0, RNG + 8, zb, 0)

    pltpu.sync_copy(list_hbm.at[t, 0], ca)
    pltpu.sync_copy(list_hbm.at[t, 1], cb)

    rows = (rows0, rows1)
    sems = (semg0, semg1)

    def gather(idxrow, u):
        pltpu.async_copy(h_hbm.at[idxrow], rows[u % 2], sems[u % 2])

    def gather_wait(idxrow, u):
        pltpu.make_async_copy(h_hbm.at[idxrow], rows[u % 2],
                              sems[u % 2]).wait()

    def accumulate(dlbuf, sub, u):
        rbuf = rows[u % 2]

        def gb(g, _):
            dlvec = dlbuf[sub, pl.ds(g * 16, 16)]
            for j in range(16):
                dl = dlvec[j]
                e = g * 16 + j
                for cg in range(8):
                    plsc.addupdate(acc.at[dl, pl.ds(cg * 16, 16)],
                                   rbuf[e, cg * 16:(cg + 1) * 16])
            return 0
        lax.fori_loop(0, 8, gb, 0)

    def pair_body(i, _):
        @pl.when(i > 0)
        def _():
            pltpu.make_async_copy(list_hbm.at[t, 2 * i], ca, semi).wait()
            pltpu.make_async_copy(list_hbm.at[t, 2 * i + 1], cb, semi).wait()

        gather(ca.at[0, 0], 0)
        for u in range(8):
            buf = ca if u < 4 else cb
            sub = u % 4
            if u < 7:
                nbuf = ca if u + 1 < 4 else cb
                gather(nbuf.at[0, (u + 1) % 4], u + 1)
            gather_wait(buf.at[0, sub], u)
            accumulate(buf.at[1], sub, u)

        @pl.when(2 * i + 3 < nblk)
        def _():
            pltpu.async_copy(list_hbm.at[t, 2 * i + 2], ca, semi)
            pltpu.async_copy(list_hbm.at[t, 2 * i + 3], cb, semi)
        return 0

    lax.fori_loop(0, nblk // 2, pair_body, 0)

    pltpu.sync_copy(acc.at[pl.ds(0, RNG)], out_hbm.at[pl.ds(lo, RNG)])


def _make_agg():
    return pl.kernel(
        _agg_body,
        out_type=jax.ShapeDtypeStruct((NP, D), jnp.float32),
        mesh=_MESH,
        scratch_types=[
            pltpu.VMEM((2, 4, 128), jnp.int32),
            pltpu.VMEM((2, 4, 128), jnp.int32),
            pltpu.VMEM((128, D), jnp.float32),
            pltpu.VMEM((128, D), jnp.float32),
            pltpu.VMEM((16,), jnp.int32),
            pltpu.VMEM((RNG + 8, D), jnp.float32),
            pltpu.SemaphoreType.DMA,
            pltpu.SemaphoreType.DMA,
            pltpu.SemaphoreType.DMA,
        ],
    )


_agg = _make_agg()


BLK = 1000
_DOT = functools.partial(
    jnp.dot, precision=lax.Precision.HIGHEST,
    preferred_element_type=jnp.float32)


def _mean(p_ref, cnt_ref):
    scale = 1.0 / jnp.maximum(cnt_ref[:, :1], 1.0)
    return p_ref[...] * scale


def _tc_layer_body(p_ref, cnt_ref, h_ref, wl_ref, wr_ref, b_ref, o_ref):
    mean = _mean(p_ref, cnt_ref)
    out = _DOT(mean, wl_ref[...]) + _DOT(h_ref[...], wr_ref[...]) + b_ref[...]
    o_ref[...] = jnp.maximum(out, 0.0)


def _tc_head_body(p_ref, cnt_ref, h_ref, wl_ref, wr_ref, b_ref,
                  wlin1_ref, blin1_ref, wlin2_ref, blin2_ref, o_ref):
    mean = _mean(p_ref, cnt_ref)
    h3 = jnp.maximum(
        _DOT(mean, wl_ref[...]) + _DOT(h_ref[...], wr_ref[...]) + b_ref[...],
        0.0)
    h4 = jnp.maximum(_DOT(h3, wlin1_ref[...]) + blin1_ref[...], 0.0)
    logits = _DOT(h4, wlin2_ref[...]) + blin2_ref[...]
    m = jnp.max(logits, axis=-1, keepdims=True)
    lse = jnp.log(jnp.sum(jnp.exp(logits - m), axis=-1, keepdims=True)) + m
    o_ref[...] = logits - lse


def _full(shape):
    return pl.BlockSpec(shape, lambda i: (0,) * len(shape))


_H_SPEC = pl.BlockSpec((BLK, D), lambda i: (i, 0))
_CNT_SPEC = pl.BlockSpec((BLK, 16), lambda i: (i, 0))

_tc_layer = pl.pallas_call(
    _tc_layer_body,
    grid=(N // BLK,),
    in_specs=[_H_SPEC, _CNT_SPEC, _H_SPEC,
              _full((D, H)), _full((D, H)), _full((1, H))],
    out_specs=_H_SPEC,
    out_shape=jax.ShapeDtypeStruct((N, H), jnp.float32),
)

_tc_head = pl.pallas_call(
    _tc_head_body,
    grid=(N // BLK,),
    in_specs=[_H_SPEC, _CNT_SPEC, _H_SPEC,
              _full((H, H)), _full((H, H)), _full((1, H)),
              _full((H, H)), _full((1, H)), _full((H, C)), _full((1, C))],
    out_specs=pl.BlockSpec((BLK, C), lambda i: (i, 0)),
    out_shape=jax.ShapeDtypeStruct((N, C), jnp.float32),
)


def kernel(x, edge_index, W1l, W1r, b1, W2l, W2r, b2, W3l, W3r, b3,
           Wlin1, blin1, Wlin2, blin2):
    src2d = edge_index[0].reshape(EROWS, 128)
    dst2d = edge_index[1].reshape(EROWS, 128)

    lists, nblk = _compact(src2d, dst2d)
    ones = jnp.ones((N, D), jnp.float32)
    cnt = _agg(ones, lists, nblk)[:N, :16]
    p1 = _agg(x, lists, nblk)[:N]
    h1 = _tc_layer(p1, cnt, x, W1l, W1r, b1.reshape(1, H))
    p2 = _agg(h1, lists, nblk)[:N]
    h2 = _tc_layer(p2, cnt, h1, W2l, W2r, b2.reshape(1, H))
    p3 = _agg(h2, lists, nblk)[:N]
    return _tc_head(p3, cnt, h2, W3l, W3r, b3.reshape(1, H),
                    Wlin1, blin1.reshape(1, H), Wlin2, blin2.reshape(1, C))
